# Initial kernel scaffold; baseline (speedup 1.0000x reference)
import jax
import jax.numpy as jnp
from jax.experimental import pallas as pl


def kernel(input_ids, token_type_ids, token_table, segment_table, position_table, ln_gamma, ln_beta):
    raise NotImplementedError("write your pallas kernel here")


# speedup vs baseline: 1.0828x; 1.0828x over previous
import functools

import jax
import jax.numpy as jnp
from jax import lax
from jax.experimental import pallas as pl
from jax.experimental.pallas import tpu as pltpu
from jax.experimental.pallas import tpu_sc as plsc

DIM = 64
EPS = 1e-5
NC = 2
NS = 16
NW = NC * NS
CHUNK = 128
NK = DIM // 16


def _rsqrt(x):
    i = lax.bitcast_convert_type(x, jnp.int32)
    i = jnp.int32(0x5F3759DF) - lax.shift_right_arithmetic(i, 1)
    y = lax.bitcast_convert_type(i, jnp.float32)
    xh = 0.5 * x
    for _ in range(2):
        y = y * (1.5 - xh * y * y)
    return y


_GATHER_DNUMS = lax.GatherDimensionNumbers(
    offset_dims=(), collapsed_slice_dims=(0,), start_index_map=(0,))


def _shuffle(v, p):
    return lax.gather(v, p[:, None], _GATHER_DNUMS, (1,),
                      mode=lax.GatherScatterMode.PROMISE_IN_BOUNDS)


def _allsum(v, perms):
    for p in perms:
        v = v + _shuffle(v, p)
    return v


@functools.lru_cache(maxsize=None)
def _build_sc_kernel(n_tokens, seq, vocab):
    tok_per_w = n_tokens // NW
    rounds = tok_per_w // CHUNK

    mesh = plsc.VectorSubcoreMesh(
        core_axis_name="c", subcore_axis_name="s", num_cores=NC, num_subcores=NS
    )

    @functools.partial(
        pl.kernel,
        out_type=jax.ShapeDtypeStruct((n_tokens, DIM), jnp.float32),
        mesh=mesh,
        scratch_types=[
            pltpu.VMEM((rounds, CHUNK), jnp.int32),
            pltpu.VMEM((rounds, CHUNK), jnp.int32),
            pltpu.VMEM((seq, DIM), jnp.float32),
            pltpu.VMEM((2, DIM), jnp.float32),
            pltpu.VMEM((DIM,), jnp.float32),
            pltpu.VMEM((DIM,), jnp.float32),
            pltpu.VMEM((2, CHUNK, DIM), jnp.float32),
            pltpu.SemaphoreType.DMA,
            pltpu.SemaphoreType.DMA,
        ],
        compiler_params=pltpu.CompilerParams(use_tc_tiling_on_sc=False),
    )
    def sc_embed(ids_hbm, tys_hbm, tok_hbm, seg_hbm, pos_hbm, g_hbm, b_hbm,
                 out_hbm, ids_v, tys_v, pos_v, seg_v, g_v, b_v, rows_v,
                 gsem0, gsem1):
        wid = lax.axis_index("s") * NC + lax.axis_index("c")
        base = wid * tok_per_w

        pltpu.sync_copy(ids_hbm.at[wid], ids_v)
        pltpu.sync_copy(tys_hbm.at[wid], tys_v)
        pltpu.sync_copy(pos_hbm, pos_v)
        pltpu.sync_copy(seg_hbm, seg_v)
        pltpu.sync_copy(g_hbm, g_v)
        pltpu.sync_copy(b_hbm, b_v)

        seg0 = [seg_v[0, pl.ds(k * 16, 16)] for k in range(NK)]
        segd = [seg_v[1, pl.ds(k * 16, 16)] - seg0[k] for k in range(NK)]
        gam = [g_v[pl.ds(k * 16, 16)] for k in range(NK)]
        bet = [b_v[pl.ds(k * 16, 16)] for k in range(NK)]

        lane = lax.iota(jnp.int32, 16)
        perms = [lax.bitwise_xor(lane, jnp.int32(m)) for m in (8, 4, 2, 1)]

        gsems = (gsem0, gsem1)

        def start_gather(g, b):
            pltpu.async_copy(tok_hbm.at[ids_v.at[g]], rows_v.at[b], gsems[b])

        def wait_gather(g, b):
            pltpu.make_async_copy(
                tok_hbm.at[ids_v.at[g]], rows_v.at[b], gsems[b]).wait()

        start_gather(0, 0)
        start_gather(1, 1)

        def do_round(g, b):
            wait_gather(g, b)
            rows = rows_v.at[b]

            def group(jg, carry):
                ty_vec = tys_v[g, pl.ds(jg * 16, 16)].astype(jnp.float32)
                for e in range(16):
                    j = jg * 16 + e
                    tyf = ty_vec[e]
                    s = lax.rem(g * CHUNK + j, seq)
                    x = []
                    for k in range(NK):
                        sl = pl.ds(k * 16, 16)
                        segk = seg0[k] + tyf * segd[k]
                        x.append(rows[j, sl] + pos_v[s, sl] + segk)
                    tot = _allsum((x[0] + x[1]) + (x[2] + x[3]), perms)
                    q = x[0] * x[0] + x[1] * x[1] + x[2] * x[2] + x[3] * x[3]
                    ssq = _allsum(q, perms)
                    mean = tot * (1.0 / DIM)
                    var = ssq * (1.0 / DIM) - mean * mean
                    rstd = _rsqrt(var + EPS)
                    m2 = mean * rstd
                    for k in range(NK):
                        y = x[k] * rstd - m2
                        rows[j, pl.ds(k * 16, 16)] = y * gam[k] + bet[k]
                return carry

            lax.fori_loop(0, CHUNK // 16, group, 0)
            pltpu.sync_copy(rows, out_hbm.at[pl.ds(base + g * CHUNK, CHUNK)])

            @pl.when(g + 2 < rounds)
            def _():
                start_gather(g + 2, b)

        def pair(gp, carry):
            do_round(2 * gp, 0)
            do_round(2 * gp + 1, 1)
            return carry

        lax.fori_loop(0, rounds // 2, pair, 0)

    return sc_embed


def kernel(input_ids, token_type_ids, token_table, segment_table,
           position_table, ln_gamma, ln_beta):
    bsz, seq = input_ids.shape
    n_tokens = bsz * seq
    vocab, dim = token_table.shape
    assert dim == DIM and n_tokens % (NW * CHUNK) == 0

    tok_per_w = n_tokens // NW
    rounds = tok_per_w // CHUNK
    ids = input_ids.reshape(NW, rounds, CHUNK).astype(jnp.int32)
    tys = token_type_ids.reshape(NW, rounds, CHUNK).astype(jnp.int32)

    sc_embed = _build_sc_kernel(n_tokens, seq, vocab)
    out = sc_embed(ids, tys, token_table, segment_table, position_table,
                   ln_gamma, ln_beta)
    return out.reshape(bsz, seq, DIM)


# speedup vs baseline: 1.0981x; 1.0141x over previous
import functools

import jax
import jax.numpy as jnp
from jax import lax
from jax.experimental import pallas as pl
from jax.experimental.pallas import tpu as pltpu
from jax.experimental.pallas import tpu_sc as plsc

DIM = 64
EPS = 1e-5
NC = 2
NS = 16
NW = NC * NS
CHUNK = 128
NK = DIM // 16


def _rsqrt(x):
    i = lax.bitcast_convert_type(x, jnp.int32)
    i = jnp.int32(0x5F3759DF) - lax.shift_right_arithmetic(i, 1)
    y = lax.bitcast_convert_type(i, jnp.float32)
    xh = 0.5 * x
    for _ in range(2):
        y = y * (1.5 - xh * y * y)
    return y


_GATHER_DNUMS = lax.GatherDimensionNumbers(
    offset_dims=(), collapsed_slice_dims=(0,), start_index_map=(0,))


def _shuffle(v, p):
    return lax.gather(v, p[:, None], _GATHER_DNUMS, (1,),
                      mode=lax.GatherScatterMode.PROMISE_IN_BOUNDS)


def _allsum(v, perms):
    for p in perms:
        v = v + _shuffle(v, p)
    return v


@functools.lru_cache(maxsize=None)
def _build_sc_kernel(n_tokens, seq, vocab):
    tok_per_w = n_tokens // NW
    rounds = tok_per_w // CHUNK

    mesh = plsc.VectorSubcoreMesh(
        core_axis_name="c", subcore_axis_name="s", num_cores=NC, num_subcores=NS
    )

    @functools.partial(
        pl.kernel,
        out_type=jax.ShapeDtypeStruct((n_tokens, DIM), jnp.float32),
        mesh=mesh,
        scratch_types=[
            pltpu.VMEM((rounds, CHUNK), jnp.int32),
            pltpu.VMEM((rounds, CHUNK), jnp.int32),
            pltpu.VMEM((seq, DIM), jnp.float32),
            pltpu.VMEM((2, DIM), jnp.float32),
            pltpu.VMEM((DIM,), jnp.float32),
            pltpu.VMEM((DIM,), jnp.float32),
            pltpu.VMEM((2, CHUNK, DIM), jnp.float32),
            pltpu.VMEM((2, CHUNK, DIM), jnp.float32),
            pltpu.SemaphoreType.DMA,
            pltpu.SemaphoreType.DMA,
            pltpu.SemaphoreType.DMA,
            pltpu.SemaphoreType.DMA,
        ],
        compiler_params=pltpu.CompilerParams(use_tc_tiling_on_sc=False),
    )
    def sc_embed(ids_hbm, tys_hbm, tok_hbm, seg_hbm, pos_hbm, g_hbm, b_hbm,
                 out_hbm, ids_v, tys_v, pos_v, seg_v, g_v, b_v, rows_v,
                 obuf_v, gsem0, gsem1, ssem0, ssem1):
        wid = lax.axis_index("s") * NC + lax.axis_index("c")
        base = wid * tok_per_w

        pltpu.sync_copy(ids_hbm.at[wid], ids_v)
        pltpu.sync_copy(tys_hbm.at[wid], tys_v)
        pltpu.sync_copy(pos_hbm, pos_v)
        pltpu.sync_copy(seg_hbm, seg_v)
        pltpu.sync_copy(g_hbm, g_v)
        pltpu.sync_copy(b_hbm, b_v)

        seg0 = [seg_v[0, pl.ds(k * 16, 16)] for k in range(NK)]
        segd = [seg_v[1, pl.ds(k * 16, 16)] - seg0[k] for k in range(NK)]
        gam = [g_v[pl.ds(k * 16, 16)] for k in range(NK)]
        bet = [b_v[pl.ds(k * 16, 16)] for k in range(NK)]

        lane = lax.iota(jnp.int32, 16)
        perms = [lax.bitwise_xor(lane, jnp.int32(m)) for m in (8, 4, 2, 1)]

        gsems = (gsem0, gsem1)
        ssems = (ssem0, ssem1)

        def start_gather(g, b):
            pltpu.async_copy(tok_hbm.at[ids_v.at[g]], rows_v.at[b], gsems[b])

        def wait_gather(g, b):
            pltpu.make_async_copy(
                tok_hbm.at[ids_v.at[g]], rows_v.at[b], gsems[b]).wait()

        def out_slice(g):
            return out_hbm.at[pl.ds(base + g * CHUNK, CHUNK)]

        def start_store(g, b):
            pltpu.async_copy(obuf_v.at[b], out_slice(g), ssems[b])

        def wait_store(g, b):
            pltpu.make_async_copy(obuf_v.at[b], out_slice(g), ssems[b]).wait()

        start_gather(0, 0)
        start_gather(1, 1)

        def do_round(g, b):
            wait_gather(g, b)

            @pl.when(g >= 2)
            def _():
                wait_store(g, b)

            rows = rows_v.at[b]
            obuf = obuf_v.at[b]

            def group(jg, carry):
                ty_vec = tys_v[g, pl.ds(jg * 16, 16)].astype(jnp.float32)
                for e in range(16):
                    j = jg * 16 + e
                    tyf = ty_vec[e]
                    s = lax.rem(g * CHUNK + j, seq)
                    x = []
                    for k in range(NK):
                        sl = pl.ds(k * 16, 16)
                        segk = seg0[k] + tyf * segd[k]
                        x.append(rows[j, sl] + pos_v[s, sl] + segk)
                    tot = _allsum((x[0] + x[1]) + (x[2] + x[3]), perms)
                    q = x[0] * x[0] + x[1] * x[1] + x[2] * x[2] + x[3] * x[3]
                    ssq = _allsum(q, perms)
                    mean = tot * (1.0 / DIM)
                    var = ssq * (1.0 / DIM) - mean * mean
                    rstd = _rsqrt(var + EPS)
                    m2 = mean * rstd
                    for k in range(NK):
                        y = x[k] * rstd - m2
                        obuf[j, pl.ds(k * 16, 16)] = y * gam[k] + bet[k]
                return carry

            lax.fori_loop(0, CHUNK // 16, group, 0)
            start_store(g, b)

            @pl.when(g + 2 < rounds)
            def _():
                start_gather(g + 2, b)

        def pair(gp, carry):
            do_round(2 * gp, 0)
            do_round(2 * gp + 1, 1)
            return carry

        lax.fori_loop(0, rounds // 2, pair, 0)
        wait_store(rounds - 2, 0)
        wait_store(rounds - 1, 1)

    return sc_embed


def kernel(input_ids, token_type_ids, token_table, segment_table,
           position_table, ln_gamma, ln_beta):
    bsz, seq = input_ids.shape
    n_tokens = bsz * seq
    vocab, dim = token_table.shape
    assert dim == DIM and n_tokens % (NW * CHUNK) == 0

    tok_per_w = n_tokens // NW
    rounds = tok_per_w // CHUNK
    ids = input_ids.reshape(NW, rounds, CHUNK).astype(jnp.int32)
    tys = token_type_ids.reshape(NW, rounds, CHUNK).astype(jnp.int32)

    sc_embed = _build_sc_kernel(n_tokens, seq, vocab)
    out = sc_embed(ids, tys, token_table, segment_table, position_table,
                   ln_gamma, ln_beta)
    return out.reshape(bsz, seq, DIM)


# speedup vs baseline: 1.2486x; 1.1370x over previous
import functools

import jax
import jax.numpy as jnp
from jax import lax
from jax.experimental import pallas as pl
from jax.experimental.pallas import tpu as pltpu
from jax.experimental.pallas import tpu_sc as plsc

DIM = 64
EPS = 1e-5
NC = 2
NS = 16
NW = NC * NS
CHUNK = 128
NBUF = 5


@functools.lru_cache(maxsize=None)
def _build_sc_gather(n_tokens, vocab):
    tok_per_w = n_tokens // NW
    rounds = tok_per_w // CHUNK
    assert rounds % NBUF == 0

    mesh = plsc.VectorSubcoreMesh(
        core_axis_name="c", subcore_axis_name="s", num_cores=NC, num_subcores=NS
    )

    @functools.partial(
        pl.kernel,
        out_type=jax.ShapeDtypeStruct((n_tokens, DIM), jnp.float32),
        mesh=mesh,
        scratch_types=[
            pltpu.VMEM((rounds, CHUNK), jnp.int32),
            pltpu.VMEM((NBUF, CHUNK, DIM), jnp.float32),
            [pltpu.SemaphoreType.DMA] * NBUF,
            [pltpu.SemaphoreType.DMA] * NBUF,
        ],
        compiler_params=pltpu.CompilerParams(use_tc_tiling_on_sc=False),
    )
    def sc_gather(ids_hbm, tok_hbm, out_hbm, ids_v, rows_v, gsems, ssems):
        wid = lax.axis_index("s") * NC + lax.axis_index("c")
        base = wid * tok_per_w

        pltpu.sync_copy(ids_hbm.at[wid], ids_v)

        def start_gather(g, b):
            pltpu.async_copy(tok_hbm.at[ids_v.at[g]], rows_v.at[b], gsems[b])

        def wait_gather(g, b):
            pltpu.make_async_copy(
                tok_hbm.at[ids_v.at[g]], rows_v.at[b], gsems[b]).wait()

        def out_slice(g):
            return out_hbm.at[pl.ds(base + g * CHUNK, CHUNK)]

        def start_store(g, b):
            pltpu.async_copy(rows_v.at[b], out_slice(g), ssems[b])

        def wait_store(g, b):
            pltpu.make_async_copy(rows_v.at[b], out_slice(g), ssems[b]).wait()

        for b in range(NBUF):
            start_gather(b, b)

        def ring(r, carry):
            for b in range(NBUF):
                g = NBUF * r + b
                wait_gather(g, b)
                start_store(g, b)

                @pl.when(g + NBUF < rounds)
                def _():
                    wait_store(g, b)
                    start_gather(g + NBUF, b)
            return carry

        lax.fori_loop(0, rounds // NBUF, ring, 0)
        for b in range(NBUF):
            wait_store(rounds - NBUF + b, b)

    return sc_gather


@functools.lru_cache(maxsize=None)
def _build_tc_ln(bsz, seq, blk_b):
    n_lines = bsz * seq // 2
    lines_per_blk = blk_b * seq // 2

    def body(g_ref, ty_ref, pos_ref, seg_ref, gam_ref, bet_ref, o_ref):
        pos = pos_ref[...]
        seg0 = seg_ref[0]
        segd = seg_ref[1] - seg0
        gam = gam_ref[...]
        bet = bet_ref[...]
        x = g_ref[...]
        tyf = ty_ref[...].astype(jnp.float32)[..., None]
        x = x + pos[None] + seg0 + tyf * segd
        mean = jnp.mean(x, axis=-1, keepdims=True)
        xc = x - mean
        var = jnp.mean(xc * xc, axis=-1, keepdims=True)
        y = xc * lax.rsqrt(var + EPS)
        o_ref[...] = y * gam + bet

    grid = bsz // blk_b
    return pl.pallas_call(
        body,
        grid=(grid,),
        in_specs=[
            pl.BlockSpec((blk_b, seq, DIM), lambda g: (g, 0, 0)),
            pl.BlockSpec((blk_b, seq), lambda g: (g, 0)),
            pl.BlockSpec((seq, DIM), lambda g: (0, 0)),
            pl.BlockSpec((2, DIM), lambda g: (0, 0)),
            pl.BlockSpec((DIM,), lambda g: (0,)),
            pl.BlockSpec((DIM,), lambda g: (0,)),
        ],
        out_specs=pl.BlockSpec((blk_b, seq, DIM), lambda g: (g, 0, 0)),
        out_shape=jax.ShapeDtypeStruct((bsz, seq, DIM), jnp.float32),
    )


def kernel(input_ids, token_type_ids, token_table, segment_table,
           position_table, ln_gamma, ln_beta):
    bsz, seq = input_ids.shape
    n_tokens = bsz * seq
    vocab, dim = token_table.shape
    assert dim == DIM and n_tokens % (NW * CHUNK) == 0 and seq % 2 == 0

    tok_per_w = n_tokens // NW
    rounds = tok_per_w // CHUNK
    ids = input_ids.reshape(NW, rounds, CHUNK).astype(jnp.int32)

    gath = _build_sc_gather(n_tokens, vocab)(ids, token_table)
    g3 = gath.reshape(bsz, seq, DIM)
    tys = token_type_ids.astype(jnp.int32)

    blk_b = 32
    out = _build_tc_ln(bsz, seq, blk_b)(
        g3, tys, position_table, segment_table, ln_gamma, ln_beta)
    return out
